# baseline (speedup 1.0000x reference)
import jax
import jax.numpy as jnp
from jax import lax
from jax.experimental import pallas as pl
from jax.experimental.pallas import tpu as pltpu
from jax.experimental.pallas import tpu_sc as plsc

MAXLEN = 200
EMBED_DIM = 64
LANES = 128
NUM_CORES = 2
NUM_SUBCORES = 16
NUM_WORKERS = NUM_CORES * NUM_SUBCORES
NJ = EMBED_DIM // 16


def _sc_body(xt_hbm, pos_hbm, tab_hbm, out_hbm,
             idx_v, pos_v, rows_v, tv,
             gsem0, gsem1, gsem2, gsem3, wsem0, wsem1):
    c = lax.axis_index("c")
    s = lax.axis_index("s")
    w = s * NUM_CORES + c

    pltpu.sync_copy(xt_hbm.at[:, pl.ds(w * LANES, LANES)], idx_v)
    pltpu.sync_copy(pos_hbm, pos_v)

    lane = lax.iota(jnp.int32, 16)
    zero16 = jnp.zeros((16,), jnp.int32)

    gsems = (gsem0, gsem1, gsem2, gsem3)
    for g in range(4):
        pltpu.async_copy(tab_hbm.at[idx_v.at[g]], rows_v.at[g], gsems[g])

    def outer(i, carry):
        for q in range(4):
            m = 4 * i + q
            b = q
            v = q & 1
            gsem = gsems[b]
            wsem = wsem0 if v == 0 else wsem1
            pltpu.make_async_copy(tab_hbm.at[idx_v.at[m]], rows_v.at[b],
                                  gsem).wait()
            @pl.when(m >= 2)
            def _():
                pltpu.make_async_copy(
                    tv.at[v],
                    out_hbm.at[m - 2, :, pl.ds(w, 1), :], wsem).wait()

            @plsc.parallel_loop(0, 16, unroll=3)
            def diag(d):
                perm = (lane + d) & 15
                for j in range(NJ):
                    e_vec = perm + 16 * j
                    s_base = ((e_vec >> 3) << 10) + ((e_vec & 7) << 7)
                    pvec = plsc.load_gather(pos_v, [zero16, m * 64 + e_vec])
                    for t0 in range(0, LANES, 16):
                        t_vec = lane + t0
                        val = plsc.load_gather(
                            rows_v.at[b], [zero16, t_vec * 64 + e_vec]) + pvec
                        plsc.store_scatter(
                            tv.at[v], [zero16, zero16, s_base + t_vec], val)

            pltpu.async_copy(tv.at[v], out_hbm.at[m, :, pl.ds(w, 1), :],
                             wsem)

            @pl.when(m + 4 < MAXLEN)
            def _():
                pltpu.async_copy(tab_hbm.at[idx_v.at[m + 4]], rows_v.at[b],
                                 gsem)
        return carry

    lax.fori_loop(0, MAXLEN // 4, outer, 0)

    pltpu.make_async_copy(tv.at[0],
                          out_hbm.at[MAXLEN - 2, :, pl.ds(w, 1), :],
                          wsem0).wait()
    pltpu.make_async_copy(tv.at[1],
                          out_hbm.at[MAXLEN - 1, :, pl.ds(w, 1), :],
                          wsem1).wait()


def kernel(x, token_table, pos_table):
    batch, maxlen = x.shape
    vocab, embed_dim = token_table.shape
    n_bblk = batch // LANES

    xt2 = x.astype(jnp.int32).T * 2
    tab2 = jnp.pad(token_table, ((0, 0), (0, 128 - embed_dim))).reshape(
        2 * vocab, embed_dim)

    mesh = plsc.VectorSubcoreMesh(core_axis_name="c", subcore_axis_name="s")
    run = pl.kernel(
        _sc_body,
        out_type=jax.ShapeDtypeStruct(
            (maxlen, embed_dim // 8, n_bblk, 8 * LANES), jnp.float32),
        mesh=mesh,
        compiler_params=pltpu.CompilerParams(
            use_tc_tiling_on_sc=False, needs_layout_passes=False),
        scratch_types=[
            pltpu.VMEM((maxlen, LANES), jnp.int32),
            pltpu.VMEM((maxlen, embed_dim), jnp.float32),
            pltpu.VMEM((4, LANES, embed_dim), jnp.float32),
            pltpu.VMEM((2, embed_dim // 8, 1, 8 * LANES), jnp.float32),
            pltpu.SemaphoreType.DMA,
            pltpu.SemaphoreType.DMA,
            pltpu.SemaphoreType.DMA,
            pltpu.SemaphoreType.DMA,
            pltpu.SemaphoreType.DMA,
            pltpu.SemaphoreType.DMA,
        ],
    )
    out5 = run(xt2, pos_table, tab2)
    return (out5.reshape(maxlen, embed_dim // 8, n_bblk, 8, LANES)
            .transpose(2, 4, 0, 1, 3).reshape(batch, maxlen, embed_dim))


# speedup vs baseline: 1.0580x; 1.0580x over previous
import jax
import jax.numpy as jnp
from jax import lax
from jax.experimental import pallas as pl
from jax.experimental.pallas import tpu as pltpu
from jax.experimental.pallas import tpu_sc as plsc

MAXLEN = 200
EMBED_DIM = 64
LANES = 128
NUM_CORES = 2
NUM_SUBCORES = 16
NUM_WORKERS = NUM_CORES * NUM_SUBCORES
NJ = EMBED_DIM // 16


def _sc_body(xt_hbm, pos_hbm, tab_hbm, out_hbm,
             idx_v, pos_v, rows_v, tv,
             gsem0, gsem1, gsem2, gsem3, wsem0, wsem1):
    c = lax.axis_index("c")
    s = lax.axis_index("s")
    w = s * NUM_CORES + c

    pltpu.sync_copy(xt_hbm.at[:, pl.ds(w * LANES, LANES)], idx_v)
    pltpu.sync_copy(pos_hbm, pos_v)

    lane = lax.iota(jnp.int32, 16)
    zero16 = jnp.zeros((16,), jnp.int32)

    gsems = (gsem0, gsem1, gsem2, gsem3)
    for g in range(4):
        pltpu.async_copy(tab_hbm.at[idx_v.at[g]], rows_v.at[g], gsems[g])

    def outer(i, carry):
        for q in range(4):
            m = 4 * i + q
            b = q
            v = q & 1
            gsem = gsems[b]
            wsem = wsem0 if v == 0 else wsem1
            pltpu.make_async_copy(tab_hbm.at[idx_v.at[m]], rows_v.at[b],
                                  gsem).wait()
            @pl.when(m >= 2)
            def _():
                pltpu.make_async_copy(
                    tv.at[v],
                    out_hbm.at[m - 2, :, pl.ds(w, 1), :], wsem).wait()

            @plsc.parallel_loop(0, 16, unroll=2)
            def diag(d):
                perm = (lane + d) & 15
                for j in range(NJ):
                    e_vec = perm + 16 * j
                    s_base = ((e_vec >> 3) << 10) + ((e_vec & 7) << 7)
                    pvec = plsc.load_gather(pos_v, [zero16, m * 64 + e_vec])
                    for t0 in range(0, LANES, 16):
                        t_vec = lane + t0
                        val = plsc.load_gather(
                            rows_v.at[b], [zero16, t_vec * 64 + e_vec]) + pvec
                        plsc.store_scatter(
                            tv.at[v], [zero16, zero16, s_base + t_vec], val)

            pltpu.async_copy(tv.at[v], out_hbm.at[m, :, pl.ds(w, 1), :],
                             wsem)

            @pl.when(m + 4 < MAXLEN)
            def _():
                pltpu.async_copy(tab_hbm.at[idx_v.at[m + 4]], rows_v.at[b],
                                 gsem)
        return carry

    lax.fori_loop(0, MAXLEN // 4, outer, 0)

    pltpu.make_async_copy(tv.at[0],
                          out_hbm.at[MAXLEN - 2, :, pl.ds(w, 1), :],
                          wsem0).wait()
    pltpu.make_async_copy(tv.at[1],
                          out_hbm.at[MAXLEN - 1, :, pl.ds(w, 1), :],
                          wsem1).wait()


def kernel(x, token_table, pos_table):
    batch, maxlen = x.shape
    vocab, embed_dim = token_table.shape
    n_bblk = batch // LANES

    xt2 = x.astype(jnp.int32).T * 2
    tab2 = jnp.pad(token_table, ((0, 0), (0, 128 - embed_dim))).reshape(
        2 * vocab, embed_dim)

    mesh = plsc.VectorSubcoreMesh(core_axis_name="c", subcore_axis_name="s")
    run = pl.kernel(
        _sc_body,
        out_type=jax.ShapeDtypeStruct(
            (maxlen, embed_dim // 8, n_bblk, 8 * LANES), jnp.float32),
        mesh=mesh,
        compiler_params=pltpu.CompilerParams(
            use_tc_tiling_on_sc=False, needs_layout_passes=False),
        scratch_types=[
            pltpu.VMEM((maxlen, LANES), jnp.int32),
            pltpu.VMEM((maxlen, embed_dim), jnp.float32),
            pltpu.VMEM((4, LANES, embed_dim), jnp.float32),
            pltpu.VMEM((2, embed_dim // 8, 1, 8 * LANES), jnp.float32),
            pltpu.SemaphoreType.DMA,
            pltpu.SemaphoreType.DMA,
            pltpu.SemaphoreType.DMA,
            pltpu.SemaphoreType.DMA,
            pltpu.SemaphoreType.DMA,
            pltpu.SemaphoreType.DMA,
        ],
    )
    out5 = run(xt2, pos_table, tab2)
    return (out5.reshape(maxlen, embed_dim // 8, n_bblk, 8, LANES)
            .transpose(2, 4, 0, 1, 3).reshape(batch, maxlen, embed_dim))
